# baseline (speedup 1.0000x reference)
import functools

import jax
import jax.numpy as jnp
from jax import lax
from jax.experimental import pallas as pl
from jax.experimental.pallas import tpu as pltpu
from jax.experimental.pallas import tpu_sc as plsc

VOCAB = 30522
HIDDEN = 768
MAX_POS = 512
B = 64
S = 512
V = 100
VIS_DIM = 2048

NC, NS = 2, 16
NW = NC * NS
CHUNK = 64

NCHUNK = 8
BC = B // NCHUNK
TOKC = BC * S

COMB = 640


def _sc_gather(table, ids, n_tok):
  tok_per_w = n_tok // NW
  cn = tok_per_w // CHUNK
  mesh = plsc.VectorSubcoreMesh(core_axis_name="c", subcore_axis_name="s")

  @functools.partial(
      pl.kernel,
      mesh=mesh,
      out_type=jax.ShapeDtypeStruct((n_tok, HIDDEN), jnp.float32),
      scratch_types=[
          pltpu.VMEM((tok_per_w,), jnp.int32),
          pltpu.VMEM((CHUNK, HIDDEN), jnp.float32),
          pltpu.VMEM((CHUNK, HIDDEN), jnp.float32),
          pltpu.SemaphoreType.DMA,
          pltpu.SemaphoreType.DMA,
          pltpu.SemaphoreType.DMA,
          pltpu.SemaphoreType.DMA,
      ],
  )
  def k(table_hbm, ids_hbm, out_hbm, idx_v, rows0, rows1, g0, g1, w0, w1):
    wid = lax.axis_index("s") * NC + lax.axis_index("c")
    base = wid * tok_per_w
    bufs = (rows0, rows1)
    gsem = (g0, g1)
    wsem = (w0, w1)
    pltpu.sync_copy(ids_hbm.at[pl.ds(base, tok_per_w)], idx_v)

    def gath(c):
      return pltpu.async_copy(
          table_hbm.at[idx_v.at[pl.ds(c * CHUNK, CHUNK)]], bufs[c % 2],
          gsem[c % 2])

    def wb(c):
      return pltpu.async_copy(
          bufs[c % 2], out_hbm.at[pl.ds(base + c * CHUNK, CHUNK)],
          wsem[c % 2])

    gd = [None] * cn
    wd = [None] * cn
    gd[0] = gath(0)
    if cn > 1:
      gd[1] = gath(1)
    for c in range(cn):
      gd[c].wait()
      wd[c] = wb(c)
      if c + 2 < cn:
        wd[c].wait()
        gd[c + 2] = gath(c + 2)
    for c in range(max(0, cn - 2), cn):
      wd[c].wait()

  return k(table, ids)


def _chunk_core(words_ref, pos_ref, type_ref, vis_ref, vtype_ref, vpos_ref,
                comb_t_ref, comb_v_ref, w_proj_ref, bias_ref, gamma_ref,
                beta_ref, out_ref):
  f32 = jnp.float32
  dn = (((0,), (0,)), ((), ()))

  def onehot_cols(pos_ids, type_ids, n):
    rows = lax.broadcasted_iota(jnp.int32, (COMB, n), 0)
    return ((rows == pos_ids) | (rows == type_ids + MAX_POS)).astype(
        jnp.bfloat16)

  oh_t = onehot_cols(pos_ref[0], type_ref[0], S)
  x_text = words_ref[...] + lax.dot_general(
      oh_t, comb_t_ref[...], dn, preferred_element_type=f32)

  oh_v = onehot_cols(vpos_ref[0], vtype_ref[0], V)
  x_vis = (jnp.dot(vis_ref[0], w_proj_ref[...],
                   preferred_element_type=f32)
           + bias_ref[...]
           + lax.dot_general(oh_v, comb_v_ref[...], dn,
                             preferred_element_type=f32))

  x = jnp.concatenate((x_text, x_vis), axis=0)
  mu = jnp.mean(x, axis=1, keepdims=True)
  xc = x - mu
  var = jnp.mean(xc * xc, axis=1, keepdims=True)
  y = xc * lax.rsqrt(var + 1e-12) * gamma_ref[...] + beta_ref[...]
  out_ref[0] = y


def _chunk_body_first(*refs):
  _chunk_core(*refs)


def _chunk_body_aliased(*refs):
  _chunk_core(*refs[:-2], refs[-1])


def kernel(input_ids, token_type_ids, position_ids, visual_embeddings,
           visual_embeddings_type, visual_position_ids,
           word_emb, pos_emb, type_emb, type_emb_visual, pos_emb_visual,
           W_proj, b_proj, ln_gamma, ln_beta):
  ids_flat = input_ids.reshape(B * S).astype(jnp.int32)
  words = [
      _sc_gather(word_emb, ids_flat[i * TOKC:(i + 1) * TOKC], TOKC)
      for i in range(NCHUNK)
  ]

  def comb(pos_t, type_t):
    pad = jnp.zeros((COMB - MAX_POS - 2, HIDDEN), jnp.float32)
    return jnp.concatenate((pos_t, type_t, pad), axis=0).astype(jnp.bfloat16)

  comb_t = comb(pos_emb, type_emb)
  comb_v = comb(pos_emb_visual, type_emb_visual)
  w_bf16 = W_proj.astype(jnp.bfloat16)
  vis_bf16 = visual_embeddings.astype(jnp.bfloat16)
  pos_3d = position_ids.astype(jnp.int32).reshape(B, 1, S)
  type_3d = token_type_ids.astype(jnp.int32).reshape(B, 1, S)
  vtype_3d = visual_embeddings_type.astype(jnp.int32).reshape(B, 1, V)
  vpos_3d = visual_position_ids.astype(jnp.int32).reshape(B, 1, V)
  bias_2d = b_proj.reshape(1, HIDDEN)
  gamma_2d = ln_gamma.reshape(1, HIDDEN)
  beta_2d = ln_beta.reshape(1, HIDDEN)

  out = None
  for i in range(NCHUNK):
    off = i * BC
    in_specs = [
        pl.BlockSpec((S, HIDDEN), lambda b: (b, 0)),
        pl.BlockSpec((1, 1, S), lambda b, o=off: (b + o, 0, 0)),
        pl.BlockSpec((1, 1, S), lambda b, o=off: (b + o, 0, 0)),
        pl.BlockSpec((1, V, VIS_DIM), lambda b, o=off: (b + o, 0, 0)),
        pl.BlockSpec((1, 1, V), lambda b, o=off: (b + o, 0, 0)),
        pl.BlockSpec((1, 1, V), lambda b, o=off: (b + o, 0, 0)),
        pl.BlockSpec((COMB, HIDDEN), lambda b: (0, 0)),
        pl.BlockSpec((COMB, HIDDEN), lambda b: (0, 0)),
        pl.BlockSpec((VIS_DIM, HIDDEN), lambda b: (0, 0)),
        pl.BlockSpec((1, HIDDEN), lambda b: (0, 0)),
        pl.BlockSpec((1, HIDDEN), lambda b: (0, 0)),
        pl.BlockSpec((1, HIDDEN), lambda b: (0, 0)),
    ]
    args = [words[i], pos_3d, type_3d, vis_bf16, vtype_3d, vpos_3d,
            comb_t, comb_v, w_bf16, bias_2d, gamma_2d, beta_2d]
    if out is None:
      body = _chunk_body_first
      aliases = {}
    else:
      in_specs.append(pl.BlockSpec(memory_space=pl.ANY))
      args.append(out)
      body = _chunk_body_aliased
      aliases = {len(args) - 1: 0}
    out = pl.pallas_call(
        body,
        grid=(BC,),
        in_specs=in_specs,
        out_specs=pl.BlockSpec((1, S + V, HIDDEN),
                               lambda b, o=off: (b + o, 0, 0)),
        out_shape=jax.ShapeDtypeStruct((B, S + V, HIDDEN), jnp.float32),
        input_output_aliases=aliases,
    )(*args)
  return out


# speedup vs baseline: 1.0981x; 1.0981x over previous
import functools

import jax
import jax.numpy as jnp
from jax import lax
from jax.experimental import pallas as pl
from jax.experimental.pallas import tpu as pltpu
from jax.experimental.pallas import tpu_sc as plsc

VOCAB = 30522
HIDDEN = 768
MAX_POS = 512
B = 64
S = 512
V = 100
VIS_DIM = 2048

NC, NS = 2, 16
NW = NC * NS
CHUNK = 64

NCHUNK = 4
BPG = 2
BC = B // NCHUNK
TOKC = BC * S

COMB = 640


def _sc_gather(table, ids, n_tok):
  tok_per_w = n_tok // NW
  cn = tok_per_w // CHUNK
  mesh = plsc.VectorSubcoreMesh(core_axis_name="c", subcore_axis_name="s")

  @functools.partial(
      pl.kernel,
      mesh=mesh,
      out_type=jax.ShapeDtypeStruct((n_tok, HIDDEN), jnp.float32),
      scratch_types=[
          pltpu.VMEM((tok_per_w,), jnp.int32),
          pltpu.VMEM((CHUNK, HIDDEN), jnp.float32),
          pltpu.VMEM((CHUNK, HIDDEN), jnp.float32),
          pltpu.SemaphoreType.DMA,
          pltpu.SemaphoreType.DMA,
          pltpu.SemaphoreType.DMA,
          pltpu.SemaphoreType.DMA,
      ],
  )
  def k(table_hbm, ids_hbm, out_hbm, idx_v, rows0, rows1, g0, g1, w0, w1):
    wid = lax.axis_index("s") * NC + lax.axis_index("c")
    base = wid * tok_per_w
    bufs = (rows0, rows1)
    gsem = (g0, g1)
    wsem = (w0, w1)
    pltpu.sync_copy(ids_hbm.at[pl.ds(base, tok_per_w)], idx_v)

    def gath(c):
      return pltpu.async_copy(
          table_hbm.at[idx_v.at[pl.ds(c * CHUNK, CHUNK)]], bufs[c % 2],
          gsem[c % 2])

    def wb(c):
      return pltpu.async_copy(
          bufs[c % 2], out_hbm.at[pl.ds(base + c * CHUNK, CHUNK)],
          wsem[c % 2])

    gd = [None] * cn
    wd = [None] * cn
    gd[0] = gath(0)
    if cn > 1:
      gd[1] = gath(1)
    for c in range(cn):
      gd[c].wait()
      wd[c] = wb(c)
      if c + 2 < cn:
        wd[c].wait()
        gd[c + 2] = gath(c + 2)
    for c in range(max(0, cn - 2), cn):
      wd[c].wait()

  return k(table, ids)


def _chunk_core(words_ref, pos_ref, type_ref, vis_ref, vtype_ref, vpos_ref,
                comb_t_ref, comb_v_ref, w_proj_ref, bias_ref, gamma_ref,
                beta_ref, out_ref):
  f32 = jnp.float32
  dn = (((0,), (0,)), ((), ()))

  def onehot_cols(pos_ids, type_ids, n):
    rows = lax.broadcasted_iota(jnp.int32, (COMB, n), 0)
    return ((rows == pos_ids) | (rows == type_ids + MAX_POS)).astype(
        jnp.bfloat16)

  for k in range(BPG):
    oh_t = onehot_cols(pos_ref[k], type_ref[k], S)
    x_text = words_ref[pl.ds(k * S, S), :] + lax.dot_general(
        oh_t, comb_t_ref[...], dn, preferred_element_type=f32)

    oh_v = onehot_cols(vpos_ref[k], vtype_ref[k], V)
    x_vis = (jnp.dot(vis_ref[k], w_proj_ref[...],
                     preferred_element_type=f32)
             + bias_ref[...]
             + lax.dot_general(oh_v, comb_v_ref[...], dn,
                               preferred_element_type=f32))

    x = jnp.concatenate((x_text, x_vis), axis=0)
    mu = jnp.mean(x, axis=1, keepdims=True)
    xc = x - mu
    var = jnp.mean(xc * xc, axis=1, keepdims=True)
    y = xc * lax.rsqrt(var + 1e-12) * gamma_ref[...] + beta_ref[...]
    out_ref[k] = y


def _chunk_body_first(*refs):
  _chunk_core(*refs)


def _chunk_body_aliased(*refs):
  _chunk_core(*refs[:-2], refs[-1])


def kernel(input_ids, token_type_ids, position_ids, visual_embeddings,
           visual_embeddings_type, visual_position_ids,
           word_emb, pos_emb, type_emb, type_emb_visual, pos_emb_visual,
           W_proj, b_proj, ln_gamma, ln_beta):
  ids_flat = input_ids.reshape(B * S).astype(jnp.int32)
  words = [
      _sc_gather(word_emb, ids_flat[i * TOKC:(i + 1) * TOKC], TOKC)
      for i in range(NCHUNK)
  ]

  def comb(pos_t, type_t):
    pad = jnp.zeros((COMB - MAX_POS - 2, HIDDEN), jnp.float32)
    return jnp.concatenate((pos_t, type_t, pad), axis=0).astype(jnp.bfloat16)

  comb_t = comb(pos_emb, type_emb)
  comb_v = comb(pos_emb_visual, type_emb_visual)
  w_bf16 = W_proj.astype(jnp.bfloat16)
  vis_bf16 = visual_embeddings.astype(jnp.bfloat16)
  pos_3d = position_ids.astype(jnp.int32).reshape(B, 1, S)
  type_3d = token_type_ids.astype(jnp.int32).reshape(B, 1, S)
  vtype_3d = visual_embeddings_type.astype(jnp.int32).reshape(B, 1, V)
  vpos_3d = visual_position_ids.astype(jnp.int32).reshape(B, 1, V)
  bias_2d = b_proj.reshape(1, HIDDEN)
  gamma_2d = ln_gamma.reshape(1, HIDDEN)
  beta_2d = ln_beta.reshape(1, HIDDEN)

  out = None
  for i in range(NCHUNK):
    off = i * BC // BPG
    in_specs = [
        pl.BlockSpec((BPG * S, HIDDEN), lambda b: (b, 0)),
        pl.BlockSpec((BPG, 1, S), lambda b, o=off: (b + o, 0, 0)),
        pl.BlockSpec((BPG, 1, S), lambda b, o=off: (b + o, 0, 0)),
        pl.BlockSpec((BPG, V, VIS_DIM), lambda b, o=off: (b + o, 0, 0)),
        pl.BlockSpec((BPG, 1, V), lambda b, o=off: (b + o, 0, 0)),
        pl.BlockSpec((BPG, 1, V), lambda b, o=off: (b + o, 0, 0)),
        pl.BlockSpec((COMB, HIDDEN), lambda b: (0, 0)),
        pl.BlockSpec((COMB, HIDDEN), lambda b: (0, 0)),
        pl.BlockSpec((VIS_DIM, HIDDEN), lambda b: (0, 0)),
        pl.BlockSpec((1, HIDDEN), lambda b: (0, 0)),
        pl.BlockSpec((1, HIDDEN), lambda b: (0, 0)),
        pl.BlockSpec((1, HIDDEN), lambda b: (0, 0)),
    ]
    args = [words[i], pos_3d, type_3d, vis_bf16, vtype_3d, vpos_3d,
            comb_t, comb_v, w_bf16, bias_2d, gamma_2d, beta_2d]
    if out is None:
      body = _chunk_body_first
      aliases = {}
    else:
      in_specs.append(pl.BlockSpec(memory_space=pl.ANY))
      args.append(out)
      body = _chunk_body_aliased
      aliases = {len(args) - 1: 0}
    out = pl.pallas_call(
        body,
        grid=(BC // BPG,),
        in_specs=in_specs,
        out_specs=pl.BlockSpec((BPG, S + V, HIDDEN),
                               lambda b, o=off: (b + o, 0, 0)),
        out_shape=jax.ShapeDtypeStruct((B, S + V, HIDDEN), jnp.float32),
        input_output_aliases=aliases,
    )(*args)
  return out
